# baseline (speedup 1.0000x reference)
import functools

import jax
import jax.numpy as jnp
from jax import lax
from jax.experimental import pallas as pl
from jax.experimental.pallas import tpu as pltpu
from jax.experimental.pallas import tpu_sc as plsc

N = 10000
C = 128
B = 4
K = 16
NPB = 2500
NPAD = 2560
NPADTOT = 10240

NCORES = 2
NSUB = 16
NW = NCORES * NSUB
NODES_W = NPAD // NW
NCH = 8
NCHUNKS = NODES_W // NCH


def _tables_body(f_ref, xyz_ref, wc_ref, wf_ref, wt_ref, bt_ref, out_ref):
    f = f_ref[...]
    out_ref[:, 0:C] = lax.dot_general(
        xyz_ref[...], wc_ref[...], (((1,), (0,)), ((), ())),
        preferred_element_type=jnp.float32)
    out_ref[:, C:2 * C] = lax.dot_general(
        f, wf_ref[...], (((1,), (0,)), ((), ())),
        preferred_element_type=jnp.float32)
    out_ref[:, 2 * C:3 * C] = lax.dot_general(
        f, wt_ref[...], (((1,), (0,)), ((), ())),
        preferred_element_type=jnp.float32) + bt_ref[...]


def _make_tables(feats_pad, xyz_pad, W_coord, W_feat, W_ft, b_ft):
    RB = 1280
    return pl.pallas_call(
        _tables_body,
        grid=(NPADTOT // RB,),
        in_specs=[
            pl.BlockSpec((RB, C), lambda i: (i, 0)),
            pl.BlockSpec((RB, 3), lambda i: (i, 0)),
            pl.BlockSpec((3, C), lambda i: (0, 0)),
            pl.BlockSpec((C, C), lambda i: (0, 0)),
            pl.BlockSpec((C, C), lambda i: (0, 0)),
            pl.BlockSpec((1, C), lambda i: (0, 0)),
        ],
        out_specs=pl.BlockSpec((RB, 3 * C), lambda i: (i, 0)),
        out_shape=jax.ShapeDtypeStruct((NPADTOT, 3 * C), jnp.float32),
    )(feats_pad, xyz_pad, W_coord, W_feat, W_ft, b_ft.reshape(1, C))


def _knn_body_b(boff, rows_ref, cols_ref, idx_ref):
    RB = rows_ref.shape[0]
    r = rows_ref[...]
    cxyz = cols_ref[...]
    xy = lax.dot_general(r, cxyz, (((1,), (0,)), ((), ())),
                         precision=lax.Precision.HIGHEST,
                         preferred_element_type=jnp.float32)
    ri = jnp.sum(r * r, axis=1, keepdims=True)
    rj = jnp.sum(cxyz * cxyz, axis=0, keepdims=True)
    d2 = (ri + rj) - (xy + xy)
    col = lax.broadcasted_iota(jnp.int32, d2.shape, 1)
    row_loc = (pl.program_id(0) * RB
               + lax.broadcasted_iota(jnp.int32, d2.shape, 0))
    d2 = jnp.where((col >= NPB) | (col == row_loc), jnp.inf, d2)
    BIG = jnp.int32(2 ** 30)
    kcol = lax.broadcasted_iota(jnp.int32, (RB, K), 1)
    row_k = (pl.program_id(0) * RB
             + lax.broadcasted_iota(jnp.int32, (RB, K), 0))
    acc = row_k + boff
    for k in range(1, K):
        m = jnp.min(d2, axis=1, keepdims=True)
        eq = d2 == m
        am = jnp.min(jnp.where(eq, col, BIG), axis=1, keepdims=True)
        d2 = jnp.where(eq, jnp.inf, d2)
        am = jnp.minimum(am, NPB - 1) + boff
        acc = jnp.where(kcol == k, am, acc)
    idx_ref[...] = acc


def _knn_idx_b(xyz_rows_b, xyz_cols_b, boff):
    RB = 128
    return pl.pallas_call(
        functools.partial(_knn_body_b, boff),
        grid=(NPAD // RB,),
        in_specs=[
            pl.BlockSpec((RB, 3), lambda i: (i, 0)),
            pl.BlockSpec((3, NPAD), lambda i: (0, 0)),
        ],
        out_specs=pl.BlockSpec((RB, K), lambda i: (i, 0)),
        out_shape=jax.ShapeDtypeStruct((NPAD, K), jnp.int32),
    )(xyz_rows_b, xyz_cols_b)


def _attn_sc_body(table_hbm, idxf_hbm, bias_hbm, out_hbm,
                  idx_v, rows0_v, rows1_v, out_v, bias_v, sem0, sem1):
    wid = lax.axis_index("s") * NCORES + lax.axis_index("c")
    pltpu.sync_copy(bias_hbm, bias_v)
    pltpu.sync_copy(idxf_hbm.at[pl.ds(wid * NODES_W * K, NODES_W * K)], idx_v)

    def start_gather(ch, rows_v, sem):
        pltpu.async_copy(
            table_hbm.at[idx_v.at[pl.ds(ch * NCH * K, NCH * K)]], rows_v, sem)

    def wait_gather(rows_v, sem):
        pltpu.make_async_copy(
            table_hbm.at[idx_v.at[pl.ds(0, NCH * K)]], rows_v, sem).wait()

    def compute(ch, rows_v):
        def node_body(n, carry2):
            for cc in range(C // 16):
                co = cc * 16
                hca = bias_v[pl.ds(co, 16)] - rows_v[n * K, pl.ds(co, 16)]
                gfa = bias_v[pl.ds(C + co, 16)] - rows_v[n * K, pl.ds(C + co, 16)]
                es = []
                for k in range(K):
                    row = n * K + k
                    l = ((rows_v[row, pl.ds(co, 16)] + hca)
                         * (rows_v[row, pl.ds(C + co, 16)] + gfa)) * 0.25
                    es.append(jnp.exp(l))
                s = es[0]
                acc = es[0] * rows_v[n * K, pl.ds(2 * C + co, 16)]
                for k in range(1, K):
                    s = s + es[k]
                    acc = acc + es[k] * rows_v[n * K + k, pl.ds(2 * C + co, 16)]
                out_v[n, pl.ds(co, 16)] = acc / s
            return carry2

        lax.fori_loop(0, NCH, node_body, 0, unroll=False)
        base = wid * NODES_W + ch * NCH
        pltpu.sync_copy(out_v, out_hbm.at[pl.ds(base, NCH)])

    start_gather(0, rows0_v, sem0)

    def pair_body(g, carry):
        ch_a = 2 * g
        start_gather(ch_a + 1, rows1_v, sem1)
        wait_gather(rows0_v, sem0)
        compute(ch_a, rows0_v)
        start_gather(jnp.minimum(ch_a + 2, NCHUNKS - 1), rows0_v, sem0)
        wait_gather(rows1_v, sem1)
        compute(ch_a + 1, rows1_v)
        return carry

    lax.fori_loop(0, NCHUNKS // 2, pair_body, 0, unroll=False)
    wait_gather(rows0_v, sem0)


_attn_sc = functools.partial(
    pl.kernel,
    mesh=plsc.VectorSubcoreMesh(core_axis_name="c", subcore_axis_name="s"),
    out_type=jax.ShapeDtypeStruct((NPAD, C), jnp.float32),
    scratch_types=[
        pltpu.VMEM((NODES_W * K,), jnp.int32),
        pltpu.VMEM((NCH * K, 3 * C), jnp.float32),
        pltpu.VMEM((NCH * K, 3 * C), jnp.float32),
        pltpu.VMEM((NCH, C), jnp.float32),
        pltpu.VMEM((2 * C,), jnp.float32),
        pltpu.SemaphoreType.DMA,
        pltpu.SemaphoreType.DMA,
    ],
)(_attn_sc_body)


def _ln_body(u_ref, f_ref, gamma_ref, beta_ref, out_ref):
    x = u_ref[...] + f_ref[...]
    mu = jnp.mean(x, axis=1, keepdims=True)
    d = x - mu
    var = jnp.mean(d * d, axis=1, keepdims=True)
    out_ref[...] = d / jnp.sqrt(var + 1e-5) * gamma_ref[...] + beta_ref[...]


def _layernorm(upd, feats, gamma, beta):
    RB = 1000
    return pl.pallas_call(
        _ln_body,
        grid=(N // RB,),
        in_specs=[
            pl.BlockSpec((RB, C), lambda i: (i, 0)),
            pl.BlockSpec((RB, C), lambda i: (i, 0)),
            pl.BlockSpec((1, C), lambda i: (0, 0)),
            pl.BlockSpec((1, C), lambda i: (0, 0)),
        ],
        out_specs=pl.BlockSpec((RB, C), lambda i: (i, 0)),
        out_shape=jax.ShapeDtypeStruct((N, C), jnp.float32),
    )(upd, feats, gamma.reshape(1, C), beta.reshape(1, C))


def kernel(features, coords, W_ft, b_ft, W_coord, b_coord, W_feat, b_feat, gamma, beta):
    xyz = coords[:, 1:4]
    feats_pad = jnp.pad(features, ((0, NPADTOT - N), (0, 0)))
    xyz_pad = jnp.pad(xyz, ((0, NPADTOT - N), (0, 0)))
    table = _make_tables(feats_pad, xyz_pad, W_coord, W_feat, W_ft, b_ft)

    xyz_b = xyz.reshape(B, NPB, 3)
    xyz_rows = jnp.pad(xyz_b, ((0, 0), (0, NPAD - NPB), (0, 0)))
    xyz_cols = jnp.transpose(xyz_rows, (0, 2, 1))
    bias_cat = jnp.concatenate([b_coord, b_feat])

    upds = []
    for b in range(B):
        idx_b = _knn_idx_b(xyz_rows[b], xyz_cols[b], b * NPB)
        upds.append(_attn_sc(table, idx_b.reshape(-1), bias_cat)[:NPB])
    upd = jnp.concatenate(upds)
    return _layernorm(upd, features, gamma, beta)


# speedup vs baseline: 1.0592x; 1.0592x over previous
import functools

import jax
import jax.numpy as jnp
from jax import lax
from jax.experimental import pallas as pl
from jax.experimental.pallas import tpu as pltpu
from jax.experimental.pallas import tpu_sc as plsc

N = 10000
C = 128
B = 4
K = 16
NPB = 2500
NPAD = 2560
NPADTOT = 10240

NCORES = 2
NSUB = 16
NW = NCORES * NSUB
NODES_W = NPAD // NW
NCH = 8
NCHUNKS = NODES_W // NCH


def _tables_body(f_ref, xyz_ref, wc_ref, wf_ref, wt_ref, bt_ref, out_ref):
    f = f_ref[...]
    out_ref[:, 0:C] = lax.dot_general(
        xyz_ref[...], wc_ref[...], (((1,), (0,)), ((), ())),
        preferred_element_type=jnp.float32)
    out_ref[:, C:2 * C] = lax.dot_general(
        f, wf_ref[...], (((1,), (0,)), ((), ())),
        preferred_element_type=jnp.float32)
    out_ref[:, 2 * C:3 * C] = lax.dot_general(
        f, wt_ref[...], (((1,), (0,)), ((), ())),
        preferred_element_type=jnp.float32) + bt_ref[...]


def _make_tables(feats_pad, xyz_pad, W_coord, W_feat, W_ft, b_ft):
    RB = 1280
    return pl.pallas_call(
        _tables_body,
        grid=(NPADTOT // RB,),
        in_specs=[
            pl.BlockSpec((RB, C), lambda i: (i, 0)),
            pl.BlockSpec((RB, 3), lambda i: (i, 0)),
            pl.BlockSpec((3, C), lambda i: (0, 0)),
            pl.BlockSpec((C, C), lambda i: (0, 0)),
            pl.BlockSpec((C, C), lambda i: (0, 0)),
            pl.BlockSpec((1, C), lambda i: (0, 0)),
        ],
        out_specs=pl.BlockSpec((RB, 3 * C), lambda i: (i, 0)),
        out_shape=jax.ShapeDtypeStruct((NPADTOT, 3 * C), jnp.float32),
    )(feats_pad, xyz_pad, W_coord, W_feat, W_ft, b_ft.reshape(1, C))


def _knn_body_b(boff, rows_ref, cols_ref, idx_ref):
    RB = rows_ref.shape[0]
    r = rows_ref[...]
    cxyz = cols_ref[...]
    dx = r[:, 0:1] - cxyz[0:1, :]
    dy = r[:, 1:2] - cxyz[1:2, :]
    dz = r[:, 2:3] - cxyz[2:3, :]
    d2 = dx * dx + dy * dy + dz * dz
    col = lax.broadcasted_iota(jnp.int32, d2.shape, 1)
    row_loc = (pl.program_id(0) * RB
               + lax.broadcasted_iota(jnp.int32, d2.shape, 0))
    d2 = jnp.where((col >= NPB) | (col == row_loc), jnp.inf, d2)
    BIG = jnp.int32(2 ** 30)
    kcol = lax.broadcasted_iota(jnp.int32, (RB, K), 1)
    row_k = (pl.program_id(0) * RB
             + lax.broadcasted_iota(jnp.int32, (RB, K), 0))
    acc = row_k + boff
    for k in range(1, K):
        m = jnp.min(d2, axis=1, keepdims=True)
        eq = d2 == m
        am = jnp.min(jnp.where(eq, col, BIG), axis=1, keepdims=True)
        d2 = jnp.where(eq, jnp.inf, d2)
        am = jnp.minimum(am, NPB - 1) + boff
        acc = jnp.where(kcol == k, am, acc)
    idx_ref[...] = acc


def _knn_idx_b(xyz_rows_b, xyz_cols_b, boff):
    RB = 256
    return pl.pallas_call(
        functools.partial(_knn_body_b, boff),
        grid=(NPAD // RB,),
        in_specs=[
            pl.BlockSpec((RB, 3), lambda i: (i, 0)),
            pl.BlockSpec((3, NPAD), lambda i: (0, 0)),
        ],
        out_specs=pl.BlockSpec((RB, K), lambda i: (i, 0)),
        out_shape=jax.ShapeDtypeStruct((NPAD, K), jnp.int32),
    )(xyz_rows_b, xyz_cols_b)


def _attn_sc_body(table_hbm, idxf_hbm, bias_hbm, out_hbm,
                  idx_v, rows0_v, rows1_v, out_v, bias_v, sem0, sem1):
    wid = lax.axis_index("s") * NCORES + lax.axis_index("c")
    pltpu.sync_copy(bias_hbm, bias_v)
    pltpu.sync_copy(idxf_hbm.at[pl.ds(wid * NODES_W * K, NODES_W * K)], idx_v)

    def start_gather(ch, rows_v, sem):
        pltpu.async_copy(
            table_hbm.at[idx_v.at[pl.ds(ch * NCH * K, NCH * K)]], rows_v, sem)

    def wait_gather(rows_v, sem):
        pltpu.make_async_copy(
            table_hbm.at[idx_v.at[pl.ds(0, NCH * K)]], rows_v, sem).wait()

    def compute(ch, rows_v):
        def node_body(n, carry2):
            for cc in range(C // 16):
                co = cc * 16
                hca = bias_v[pl.ds(co, 16)] - rows_v[n * K, pl.ds(co, 16)]
                gfa = bias_v[pl.ds(C + co, 16)] - rows_v[n * K, pl.ds(C + co, 16)]
                es = []
                for k in range(K):
                    row = n * K + k
                    l = ((rows_v[row, pl.ds(co, 16)] + hca)
                         * (rows_v[row, pl.ds(C + co, 16)] + gfa)) * 0.25
                    es.append(jnp.exp(l))
                s = es[0]
                acc = es[0] * rows_v[n * K, pl.ds(2 * C + co, 16)]
                for k in range(1, K):
                    s = s + es[k]
                    acc = acc + es[k] * rows_v[n * K + k, pl.ds(2 * C + co, 16)]
                out_v[n, pl.ds(co, 16)] = acc / s
            return carry2

        lax.fori_loop(0, NCH, node_body, 0, unroll=False)
        base = wid * NODES_W + ch * NCH
        pltpu.sync_copy(out_v, out_hbm.at[pl.ds(base, NCH)])

    start_gather(0, rows0_v, sem0)

    def pair_body(g, carry):
        ch_a = 2 * g
        start_gather(ch_a + 1, rows1_v, sem1)
        wait_gather(rows0_v, sem0)
        compute(ch_a, rows0_v)
        start_gather(jnp.minimum(ch_a + 2, NCHUNKS - 1), rows0_v, sem0)
        wait_gather(rows1_v, sem1)
        compute(ch_a + 1, rows1_v)
        return carry

    lax.fori_loop(0, NCHUNKS // 2, pair_body, 0, unroll=False)
    wait_gather(rows0_v, sem0)


_attn_sc = functools.partial(
    pl.kernel,
    mesh=plsc.VectorSubcoreMesh(core_axis_name="c", subcore_axis_name="s"),
    out_type=jax.ShapeDtypeStruct((NPAD, C), jnp.float32),
    scratch_types=[
        pltpu.VMEM((NODES_W * K,), jnp.int32),
        pltpu.VMEM((NCH * K, 3 * C), jnp.float32),
        pltpu.VMEM((NCH * K, 3 * C), jnp.float32),
        pltpu.VMEM((NCH, C), jnp.float32),
        pltpu.VMEM((2 * C,), jnp.float32),
        pltpu.SemaphoreType.DMA,
        pltpu.SemaphoreType.DMA,
    ],
)(_attn_sc_body)


def _ln_body(u_ref, f_ref, gamma_ref, beta_ref, out_ref):
    x = u_ref[...] + f_ref[...]
    mu = jnp.mean(x, axis=1, keepdims=True)
    d = x - mu
    var = jnp.mean(d * d, axis=1, keepdims=True)
    out_ref[...] = d / jnp.sqrt(var + 1e-5) * gamma_ref[...] + beta_ref[...]


def _layernorm(upd, feats, gamma, beta):
    RB = 1000
    return pl.pallas_call(
        _ln_body,
        grid=(N // RB,),
        in_specs=[
            pl.BlockSpec((RB, C), lambda i: (i, 0)),
            pl.BlockSpec((RB, C), lambda i: (i, 0)),
            pl.BlockSpec((1, C), lambda i: (0, 0)),
            pl.BlockSpec((1, C), lambda i: (0, 0)),
        ],
        out_specs=pl.BlockSpec((RB, C), lambda i: (i, 0)),
        out_shape=jax.ShapeDtypeStruct((N, C), jnp.float32),
    )(upd, feats, gamma.reshape(1, C), beta.reshape(1, C))


def kernel(features, coords, W_ft, b_ft, W_coord, b_coord, W_feat, b_feat, gamma, beta):
    xyz = coords[:, 1:4]
    feats_pad = jnp.pad(features, ((0, NPADTOT - N), (0, 0)))
    xyz_pad = jnp.pad(xyz, ((0, NPADTOT - N), (0, 0)))
    table = _make_tables(feats_pad, xyz_pad, W_coord, W_feat, W_ft, b_ft)

    xyz_b = xyz.reshape(B, NPB, 3)
    xyz_rows = jnp.pad(xyz_b, ((0, 0), (0, NPAD - NPB), (0, 0)))
    xyz_cols = jnp.transpose(xyz_rows, (0, 2, 1))
    bias_cat = jnp.concatenate([b_coord, b_feat])

    upds = []
    for b in range(B):
        idx_b = _knn_idx_b(xyz_rows[b], xyz_cols[b], b * NPB)
        upds.append(_attn_sc(table, idx_b.reshape(-1), bias_cat)[:NPB])
    upd = jnp.concatenate(upds)
    return _layernorm(upd, features, gamma, beta)


# speedup vs baseline: 1.0806x; 1.0202x over previous
import functools

import jax
import jax.numpy as jnp
from jax import lax
from jax.experimental import pallas as pl
from jax.experimental.pallas import tpu as pltpu
from jax.experimental.pallas import tpu_sc as plsc

N = 10000
C = 128
B = 4
K = 16
NPB = 2500
NPAD = 2560
NPADTOT = 10240

NCORES = 2
NSUB = 16
NW = NCORES * NSUB
UNIT = NPAD // 2
NODES_W = UNIT // NW
NCH = 8
NCHUNKS = NODES_W // NCH


def _tables_body(f_ref, xyz_ref, wc_ref, wf_ref, wt_ref, bt_ref, out_ref):
    f = f_ref[...]
    out_ref[:, 0:C] = lax.dot_general(
        xyz_ref[...], wc_ref[...], (((1,), (0,)), ((), ())),
        preferred_element_type=jnp.float32)
    out_ref[:, C:2 * C] = lax.dot_general(
        f, wf_ref[...], (((1,), (0,)), ((), ())),
        preferred_element_type=jnp.float32)
    out_ref[:, 2 * C:3 * C] = lax.dot_general(
        f, wt_ref[...], (((1,), (0,)), ((), ())),
        preferred_element_type=jnp.float32) + bt_ref[...]


def _make_tables(feats_pad, xyz_pad, W_coord, W_feat, W_ft, b_ft):
    RB = 1280
    return pl.pallas_call(
        _tables_body,
        grid=(NPADTOT // RB,),
        in_specs=[
            pl.BlockSpec((RB, C), lambda i: (i, 0)),
            pl.BlockSpec((RB, 3), lambda i: (i, 0)),
            pl.BlockSpec((3, C), lambda i: (0, 0)),
            pl.BlockSpec((C, C), lambda i: (0, 0)),
            pl.BlockSpec((C, C), lambda i: (0, 0)),
            pl.BlockSpec((1, C), lambda i: (0, 0)),
        ],
        out_specs=pl.BlockSpec((RB, 3 * C), lambda i: (i, 0)),
        out_shape=jax.ShapeDtypeStruct((NPADTOT, 3 * C), jnp.float32),
    )(feats_pad, xyz_pad, W_coord, W_feat, W_ft, b_ft.reshape(1, C))


def _knn_body_b(boff, roff, rows_ref, cols_ref, idx_ref):
    RB = rows_ref.shape[0]
    r = rows_ref[...]
    cxyz = cols_ref[...]
    dx = r[:, 0:1] - cxyz[0:1, :]
    dy = r[:, 1:2] - cxyz[1:2, :]
    dz = r[:, 2:3] - cxyz[2:3, :]
    d2 = dx * dx + dy * dy + dz * dz
    col = lax.broadcasted_iota(jnp.int32, d2.shape, 1)
    row_loc = (roff + pl.program_id(0) * RB
               + lax.broadcasted_iota(jnp.int32, d2.shape, 0))
    d2 = jnp.where((col >= NPB) | (col == row_loc), jnp.inf, d2)
    BIG = jnp.int32(2 ** 30)
    kcol = lax.broadcasted_iota(jnp.int32, (RB, K), 1)
    row_k = (roff + pl.program_id(0) * RB
             + lax.broadcasted_iota(jnp.int32, (RB, K), 0))
    acc = row_k + boff
    for k in range(1, K):
        m = jnp.min(d2, axis=1, keepdims=True)
        eq = d2 == m
        am = jnp.min(jnp.where(eq, col, BIG), axis=1, keepdims=True)
        d2 = jnp.where(eq, jnp.inf, d2)
        am = jnp.minimum(am, NPB - 1) + boff
        acc = jnp.where(kcol == k, am, acc)
    idx_ref[...] = acc


def _knn_idx_b(xyz_rows_u, xyz_cols_b, boff, roff):
    RB = 128
    return pl.pallas_call(
        functools.partial(_knn_body_b, boff, roff),
        grid=(UNIT // RB,),
        in_specs=[
            pl.BlockSpec((RB, 3), lambda i: (i, 0)),
            pl.BlockSpec((3, NPAD), lambda i: (0, 0)),
        ],
        out_specs=pl.BlockSpec((RB, K), lambda i: (i, 0)),
        out_shape=jax.ShapeDtypeStruct((UNIT, K), jnp.int32),
    )(xyz_rows_u, xyz_cols_b)


def _attn_sc_body(table_hbm, idxf_hbm, bias_hbm, out_hbm,
                  idx_v, rows0_v, rows1_v, out_v, bias_v, sem0, sem1):
    wid = lax.axis_index("s") * NCORES + lax.axis_index("c")
    pltpu.sync_copy(bias_hbm, bias_v)
    pltpu.sync_copy(idxf_hbm.at[pl.ds(wid * NODES_W * K, NODES_W * K)], idx_v)

    def start_gather(ch, rows_v, sem):
        pltpu.async_copy(
            table_hbm.at[idx_v.at[pl.ds(ch * NCH * K, NCH * K)]], rows_v, sem)

    def wait_gather(rows_v, sem):
        pltpu.make_async_copy(
            table_hbm.at[idx_v.at[pl.ds(0, NCH * K)]], rows_v, sem).wait()

    def compute(ch, rows_v):
        def node_body(n, carry2):
            for cc in range(C // 16):
                co = cc * 16
                hca = bias_v[pl.ds(co, 16)] - rows_v[n * K, pl.ds(co, 16)]
                gfa = bias_v[pl.ds(C + co, 16)] - rows_v[n * K, pl.ds(C + co, 16)]
                es = []
                for k in range(K):
                    row = n * K + k
                    l = ((rows_v[row, pl.ds(co, 16)] + hca)
                         * (rows_v[row, pl.ds(C + co, 16)] + gfa)) * 0.25
                    es.append(jnp.exp(l))
                s = es[0]
                acc = es[0] * rows_v[n * K, pl.ds(2 * C + co, 16)]
                for k in range(1, K):
                    s = s + es[k]
                    acc = acc + es[k] * rows_v[n * K + k, pl.ds(2 * C + co, 16)]
                out_v[n, pl.ds(co, 16)] = acc / s
            return carry2

        lax.fori_loop(0, NCH, node_body, 0, unroll=False)
        base = wid * NODES_W + ch * NCH
        pltpu.sync_copy(out_v, out_hbm.at[pl.ds(base, NCH)])

    start_gather(0, rows0_v, sem0)

    def pair_body(g, carry):
        ch_a = 2 * g
        start_gather(ch_a + 1, rows1_v, sem1)
        wait_gather(rows0_v, sem0)
        compute(ch_a, rows0_v)
        start_gather(ch_a + 2, rows0_v, sem0)
        wait_gather(rows1_v, sem1)
        compute(ch_a + 1, rows1_v)
        return carry

    lax.fori_loop(0, NCHUNKS // 2, pair_body, 0, unroll=False)
    wait_gather(rows0_v, sem0)
    compute(NCHUNKS - 1, rows0_v)


_attn_sc = functools.partial(
    pl.kernel,
    mesh=plsc.VectorSubcoreMesh(core_axis_name="c", subcore_axis_name="s"),
    out_type=jax.ShapeDtypeStruct((UNIT, C), jnp.float32),
    scratch_types=[
        pltpu.VMEM((NODES_W * K,), jnp.int32),
        pltpu.VMEM((NCH * K, 3 * C), jnp.float32),
        pltpu.VMEM((NCH * K, 3 * C), jnp.float32),
        pltpu.VMEM((NCH, C), jnp.float32),
        pltpu.VMEM((2 * C,), jnp.float32),
        pltpu.SemaphoreType.DMA,
        pltpu.SemaphoreType.DMA,
    ],
)(_attn_sc_body)


def _ln_body(u_ref, f_ref, gamma_ref, beta_ref, out_ref):
    x = u_ref[...] + f_ref[...]
    mu = jnp.mean(x, axis=1, keepdims=True)
    d = x - mu
    var = jnp.mean(d * d, axis=1, keepdims=True)
    out_ref[...] = d / jnp.sqrt(var + 1e-5) * gamma_ref[...] + beta_ref[...]


def _layernorm(upd, feats, gamma, beta):
    RB = 1000
    return pl.pallas_call(
        _ln_body,
        grid=(N // RB,),
        in_specs=[
            pl.BlockSpec((RB, C), lambda i: (i, 0)),
            pl.BlockSpec((RB, C), lambda i: (i, 0)),
            pl.BlockSpec((1, C), lambda i: (0, 0)),
            pl.BlockSpec((1, C), lambda i: (0, 0)),
        ],
        out_specs=pl.BlockSpec((RB, C), lambda i: (i, 0)),
        out_shape=jax.ShapeDtypeStruct((N, C), jnp.float32),
    )(upd, feats, gamma.reshape(1, C), beta.reshape(1, C))


def kernel(features, coords, W_ft, b_ft, W_coord, b_coord, W_feat, b_feat, gamma, beta):
    xyz = coords[:, 1:4]
    feats_pad = jnp.pad(features, ((0, NPADTOT - N), (0, 0)))
    xyz_pad = jnp.pad(xyz, ((0, NPADTOT - N), (0, 0)))
    table = _make_tables(feats_pad, xyz_pad, W_coord, W_feat, W_ft, b_ft)

    xyz_b = xyz.reshape(B, NPB, 3)
    xyz_rows = jnp.pad(xyz_b, ((0, 0), (0, NPAD - NPB), (0, 0)))
    xyz_cols = jnp.transpose(xyz_rows, (0, 2, 1))
    bias_cat = jnp.concatenate([b_coord, b_feat])

    upds = []
    for b in range(B):
        for h in range(2):
            roff = h * UNIT
            idx_u = _knn_idx_b(
                lax.slice_in_dim(xyz_rows[b], roff, roff + UNIT),
                xyz_cols[b], b * NPB, roff)
            u = _attn_sc(table, idx_u.reshape(-1), bias_cat)
            upds.append(u if h == 0 else u[:NPB - UNIT])
    upd = jnp.concatenate(upds)
    return _layernorm(upd, features, gamma, beta)


# speedup vs baseline: 1.0970x; 1.0152x over previous
import functools

import jax
import jax.numpy as jnp
from jax import lax
from jax.experimental import pallas as pl
from jax.experimental.pallas import tpu as pltpu
from jax.experimental.pallas import tpu_sc as plsc

N = 10000
C = 128
B = 4
K = 16
NPB = 2500
NPAD = 2560
NPADTOT = 10240

NCORES = 2
NSUB = 16
NW = NCORES * NSUB
UNIT = NPAD
NODES_W = UNIT // NW
NCH = 8
NCHUNKS = NODES_W // NCH


def _tables_body(f_ref, xyz_ref, wc_ref, wf_ref, wt_ref, bt_ref, out_ref):
    f = f_ref[...]
    out_ref[:, 0:C] = lax.dot_general(
        xyz_ref[...], wc_ref[...], (((1,), (0,)), ((), ())),
        preferred_element_type=jnp.float32)
    out_ref[:, C:2 * C] = lax.dot_general(
        f, wf_ref[...], (((1,), (0,)), ((), ())),
        preferred_element_type=jnp.float32)
    out_ref[:, 2 * C:3 * C] = lax.dot_general(
        f, wt_ref[...], (((1,), (0,)), ((), ())),
        preferred_element_type=jnp.float32) + bt_ref[...]


def _make_tables(feats_pad, xyz_pad, W_coord, W_feat, W_ft, b_ft):
    RB = 1280
    return pl.pallas_call(
        _tables_body,
        grid=(NPADTOT // RB,),
        in_specs=[
            pl.BlockSpec((RB, C), lambda i: (i, 0)),
            pl.BlockSpec((RB, 3), lambda i: (i, 0)),
            pl.BlockSpec((3, C), lambda i: (0, 0)),
            pl.BlockSpec((C, C), lambda i: (0, 0)),
            pl.BlockSpec((C, C), lambda i: (0, 0)),
            pl.BlockSpec((1, C), lambda i: (0, 0)),
        ],
        out_specs=pl.BlockSpec((RB, 3 * C), lambda i: (i, 0)),
        out_shape=jax.ShapeDtypeStruct((NPADTOT, 3 * C), jnp.float32),
    )(feats_pad, xyz_pad, W_coord, W_feat, W_ft, b_ft.reshape(1, C))


def _knn_body_b(boff, roff, rows_ref, cols_ref, idx_ref):
    RB = rows_ref.shape[0]
    r = rows_ref[...]
    cxyz = cols_ref[...]
    dx = r[:, 0:1] - cxyz[0:1, :]
    dy = r[:, 1:2] - cxyz[1:2, :]
    dz = r[:, 2:3] - cxyz[2:3, :]
    d2 = dx * dx + dy * dy + dz * dz
    col = lax.broadcasted_iota(jnp.int32, d2.shape, 1)
    row_loc = (roff + pl.program_id(0) * RB
               + lax.broadcasted_iota(jnp.int32, d2.shape, 0))
    d2 = jnp.where((col >= NPB) | (col == row_loc), jnp.inf, d2)
    BIG = jnp.int32(2 ** 30)
    kcol = lax.broadcasted_iota(jnp.int32, (RB, K), 1)
    row_k = (roff + pl.program_id(0) * RB
             + lax.broadcasted_iota(jnp.int32, (RB, K), 0))
    acc = row_k + boff
    for k in range(1, K):
        m = jnp.min(d2, axis=1, keepdims=True)
        eq = d2 == m
        am = jnp.min(jnp.where(eq, col, BIG), axis=1, keepdims=True)
        d2 = jnp.where(eq, jnp.inf, d2)
        am = jnp.minimum(am, NPB - 1) + boff
        acc = jnp.where(kcol == k, am, acc)
    idx_ref[...] = acc


def _knn_idx_b(xyz_rows_u, xyz_cols_b, boff, roff):
    RB = 128
    return pl.pallas_call(
        functools.partial(_knn_body_b, boff, roff),
        grid=(UNIT // RB,),
        in_specs=[
            pl.BlockSpec((RB, 3), lambda i: (i, 0)),
            pl.BlockSpec((3, NPAD), lambda i: (0, 0)),
        ],
        out_specs=pl.BlockSpec((RB, K), lambda i: (i, 0)),
        out_shape=jax.ShapeDtypeStruct((UNIT, K), jnp.int32),
    )(xyz_rows_u, xyz_cols_b)


def _attn_sc_body(table_hbm, idxf_hbm, bias_hbm, out_hbm,
                  idx_v, rows0_v, rows1_v, out_v, bias_v, sem0, sem1):
    wid = lax.axis_index("s") * NCORES + lax.axis_index("c")
    pltpu.sync_copy(bias_hbm, bias_v)
    pltpu.sync_copy(idxf_hbm.at[pl.ds(wid * NODES_W * K, NODES_W * K)], idx_v)

    def start_gather(ch, rows_v, sem):
        pltpu.async_copy(
            table_hbm.at[idx_v.at[pl.ds(ch * NCH * K, NCH * K)]], rows_v, sem)

    def wait_gather(rows_v, sem):
        pltpu.make_async_copy(
            table_hbm.at[idx_v.at[pl.ds(0, NCH * K)]], rows_v, sem).wait()

    def compute(ch, rows_v):
        def node_body(n, carry2):
            for cc in range(C // 16):
                co = cc * 16
                hca = bias_v[pl.ds(co, 16)] - rows_v[n * K, pl.ds(co, 16)]
                gfa = bias_v[pl.ds(C + co, 16)] - rows_v[n * K, pl.ds(C + co, 16)]
                es = []
                for k in range(K):
                    row = n * K + k
                    l = ((rows_v[row, pl.ds(co, 16)] + hca)
                         * (rows_v[row, pl.ds(C + co, 16)] + gfa)) * 0.25
                    es.append(jnp.exp(l))
                s = es[0]
                acc = es[0] * rows_v[n * K, pl.ds(2 * C + co, 16)]
                for k in range(1, K):
                    s = s + es[k]
                    acc = acc + es[k] * rows_v[n * K + k, pl.ds(2 * C + co, 16)]
                out_v[n, pl.ds(co, 16)] = acc / s
            return carry2

        lax.fori_loop(0, NCH, node_body, 0, unroll=False)
        base = wid * NODES_W + ch * NCH
        pltpu.sync_copy(out_v, out_hbm.at[pl.ds(base, NCH)])

    start_gather(0, rows0_v, sem0)

    def pair_body(g, carry):
        ch_a = 2 * g
        start_gather(ch_a + 1, rows1_v, sem1)
        wait_gather(rows0_v, sem0)
        compute(ch_a, rows0_v)
        start_gather(jnp.minimum(ch_a + 2, NCHUNKS - 1), rows0_v, sem0)
        wait_gather(rows1_v, sem1)
        compute(ch_a + 1, rows1_v)
        return carry

    lax.fori_loop(0, NCHUNKS // 2, pair_body, 0, unroll=False)
    wait_gather(rows0_v, sem0)


_attn_sc = functools.partial(
    pl.kernel,
    mesh=plsc.VectorSubcoreMesh(core_axis_name="c", subcore_axis_name="s"),
    out_type=jax.ShapeDtypeStruct((UNIT, C), jnp.float32),
    scratch_types=[
        pltpu.VMEM((NODES_W * K,), jnp.int32),
        pltpu.VMEM((NCH * K, 3 * C), jnp.float32),
        pltpu.VMEM((NCH * K, 3 * C), jnp.float32),
        pltpu.VMEM((NCH, C), jnp.float32),
        pltpu.VMEM((2 * C,), jnp.float32),
        pltpu.SemaphoreType.DMA,
        pltpu.SemaphoreType.DMA,
    ],
)(_attn_sc_body)


def _ln_body(u_ref, f_ref, gamma_ref, beta_ref, out_ref):
    x = u_ref[...] + f_ref[...]
    mu = jnp.mean(x, axis=1, keepdims=True)
    d = x - mu
    var = jnp.mean(d * d, axis=1, keepdims=True)
    out_ref[...] = d / jnp.sqrt(var + 1e-5) * gamma_ref[...] + beta_ref[...]


def _layernorm(upd, feats, gamma, beta):
    RB = 1000
    return pl.pallas_call(
        _ln_body,
        grid=(N // RB,),
        in_specs=[
            pl.BlockSpec((RB, C), lambda i: (i, 0)),
            pl.BlockSpec((RB, C), lambda i: (i, 0)),
            pl.BlockSpec((1, C), lambda i: (0, 0)),
            pl.BlockSpec((1, C), lambda i: (0, 0)),
        ],
        out_specs=pl.BlockSpec((RB, C), lambda i: (i, 0)),
        out_shape=jax.ShapeDtypeStruct((N, C), jnp.float32),
    )(upd, feats, gamma.reshape(1, C), beta.reshape(1, C))


def kernel(features, coords, W_ft, b_ft, W_coord, b_coord, W_feat, b_feat, gamma, beta):
    xyz = coords[:, 1:4]
    feats_pad = jnp.pad(features, ((0, NPADTOT - N), (0, 0)))
    xyz_pad = jnp.pad(xyz, ((0, NPADTOT - N), (0, 0)))
    table = _make_tables(feats_pad, xyz_pad, W_coord, W_feat, W_ft, b_ft)

    xyz_b = xyz.reshape(B, NPB, 3)
    xyz_rows = jnp.pad(xyz_b, ((0, 0), (0, NPAD - NPB), (0, 0)))
    xyz_cols = jnp.transpose(xyz_rows, (0, 2, 1))
    bias_cat = jnp.concatenate([b_coord, b_feat])

    upds = []
    for b in range(B):
        idx_b = _knn_idx_b(xyz_rows[b], xyz_cols[b], b * NPB, 0)
        upds.append(_attn_sc(table, idx_b.reshape(-1), bias_cat)[:NPB])
    upd = jnp.concatenate(upds)
    return _layernorm(upd, features, gamma, beta)
